# Initial kernel scaffold; baseline (speedup 1.0000x reference)
import jax
import jax.numpy as jnp
from jax.experimental import pallas as pl


def kernel(x, expert_embeddings, W1, b1, W2, b2):
    raise NotImplementedError("write your pallas kernel here")


# speedup vs baseline: 4.6561x; 4.6561x over previous
import functools

import jax
import jax.numpy as jnp
from jax.experimental import pallas as pl
from jax.experimental.pallas import tpu as pltpu

TOPK = 2


def _moe_body(x_ref, embT_ref, W1_ref, b1_ref, W2_ref, b2_ref, out_ref):
    e = pl.program_id(1)
    x = x_ref[...]
    logits = jnp.dot(x, embT_ref[...], preferred_element_type=jnp.float32)
    E = logits.shape[1]
    iota = jax.lax.broadcasted_iota(jnp.int32, logits.shape, 1)
    m0 = jnp.max(logits, axis=1, keepdims=True)
    i0 = jnp.min(jnp.where(logits == m0, iota, E), axis=1, keepdims=True)
    l2 = jnp.where(iota == i0, -jnp.inf, logits)
    m1 = jnp.max(l2, axis=1, keepdims=True)
    i1 = jnp.min(jnp.where(l2 == m1, iota, E), axis=1, keepdims=True)
    e1 = jnp.exp(m1 - m0)
    s = 1.0 + e1
    w0 = 1.0 / s
    w1 = e1 / s
    w_e = (jnp.where(i0 == e, w0, 0.0) + jnp.where(i1 == e, w1, 0.0))

    h = jnp.dot(x, W1_ref[0], preferred_element_type=jnp.float32) + b1_ref[0]
    h = jnp.maximum(h, 0.0)
    y = jnp.dot(h, W2_ref[0], preferred_element_type=jnp.float32) + b2_ref[0]
    y = y * w_e

    @pl.when(e == 0)
    def _():
        out_ref[...] = y

    @pl.when(e > 0)
    def _():
        out_ref[...] += y


def kernel(x, expert_embeddings, W1, b1, W2, b2):
    B, S, D = x.shape
    E, _, F = W1.shape
    N = B * S
    TB = 512
    x2 = x.reshape(N, D)
    embT = expert_embeddings.T
    b1r = b1.reshape(E, 1, F)
    b2r = b2.reshape(E, 1, D)

    out = pl.pallas_call(
        _moe_body,
        grid=(N // TB, E),
        in_specs=[
            pl.BlockSpec((TB, D), lambda t, e: (t, 0)),
            pl.BlockSpec((D, E), lambda t, e: (0, 0)),
            pl.BlockSpec((1, D, F), lambda t, e: (e, 0, 0)),
            pl.BlockSpec((1, 1, F), lambda t, e: (e, 0, 0)),
            pl.BlockSpec((1, F, D), lambda t, e: (e, 0, 0)),
            pl.BlockSpec((1, 1, D), lambda t, e: (e, 0, 0)),
        ],
        out_specs=pl.BlockSpec((TB, D), lambda t, e: (t, 0)),
        out_shape=jax.ShapeDtypeStruct((N, D), jnp.float32),
        compiler_params=pltpu.CompilerParams(
            dimension_semantics=("parallel", "arbitrary"),
        ),
    )(x2, embT, W1, b1r, W2, b2r)
    return out.reshape(B, S, D)
